# baseline (speedup 1.0000x reference)
import functools

import jax
import jax.numpy as jnp
from jax import lax
from jax.experimental import pallas as pl
from jax.experimental.pallas import tpu as pltpu
from jax.experimental.pallas import tpu_sc as plsc

VOCAB = 1000
VOCAB_PAD = 1024
EMBED_DIM = 32
BATCH = 16384
PAD_DIM = 128
LANES = 16
QUARTERS = PAD_DIM // EMBED_DIM


@functools.lru_cache(maxsize=None)
def _build():
    info = plsc.get_sparse_core_info()
    nc, ns = info.num_cores, info.num_subcores
    nw = nc * ns
    lines = BATCH // QUARTERS
    l_per_w = lines // nw
    q_stride = lines

    mesh = plsc.VectorSubcoreMesh(core_axis_name="c", subcore_axis_name="s")

    @functools.partial(
        pl.kernel,
        mesh=mesh,
        out_type=jax.ShapeDtypeStruct((lines, PAD_DIM), jnp.float32),
        scratch_types=[
            pltpu.VMEM((QUARTERS * l_per_w,), jnp.int32),
            pltpu.VMEM((QUARTERS, l_per_w, PAD_DIM), jnp.float32),
            pltpu.VMEM((l_per_w, PAD_DIM), jnp.float32),
            pltpu.VMEM_SHARED((VOCAB_PAD, PAD_DIM), jnp.float32),
            pltpu.SemaphoreType.DMA,
        ],
    )
    def gather_kernel(idx_hbm, table_hbm, out_hbm, idx_v, bufs, packed,
                      table_sp, gsem):
        sid = lax.axis_index("s")
        wid = sid * nc + lax.axis_index("c")
        lbase = wid * l_per_w
        @pl.when(sid < 8)
        def _():
            pltpu.sync_copy(
                table_hbm.at[pl.ds(sid * (VOCAB_PAD // 8), VOCAB_PAD // 8)],
                table_sp.at[pl.ds(sid * (VOCAB_PAD // 8), VOCAB_PAD // 8)])
        for q in range(QUARTERS):
            pltpu.sync_copy(
                idx_hbm.at[pl.ds(q * q_stride + lbase, l_per_w)],
                idx_v.at[pl.ds(q * l_per_w, l_per_w)])
        plsc.subcore_barrier()

        copies = [
            pltpu.async_copy(
                table_sp.at[idx_v.at[pl.ds(q * l_per_w, l_per_w)]],
                bufs.at[q], gsem)
            for q in range(QUARTERS)
        ]
        for c in copies:
            c.wait()

        def pack(l, carry):
            for q in range(QUARTERS):
                for h in range(EMBED_DIM // LANES):
                    packed[l, pl.ds(q * EMBED_DIM + h * LANES, LANES)] = (
                        bufs[q, l, pl.ds(h * LANES, LANES)])
            return carry

        lax.fori_loop(0, l_per_w, pack, 0)
        pltpu.sync_copy(packed, out_hbm.at[pl.ds(lbase, l_per_w)])

    return gather_kernel


def kernel(user_id, table):
    table_padded = jnp.pad(
        table, ((0, VOCAB_PAD - VOCAB), (0, PAD_DIM - EMBED_DIM)))
    out_packed = _build()(user_id, table_padded)
    return jnp.concatenate(
        [out_packed[:, q * EMBED_DIM:(q + 1) * EMBED_DIM]
         for q in range(QUARTERS)], axis=0)


# speedup vs baseline: 1.1625x; 1.1625x over previous
import functools

import jax
import jax.numpy as jnp
from jax import lax
from jax.experimental import pallas as pl
from jax.experimental.pallas import tpu as pltpu
from jax.experimental.pallas import tpu_sc as plsc

VOCAB = 1000
VOCAB_PAD = 1024
EMBED_DIM = 32
BATCH = 16384
PAD_DIM = 128
CHUNK = 64


@functools.lru_cache(maxsize=None)
def _build(offset, nrows):
    info = plsc.get_sparse_core_info()
    nc, ns = info.num_cores, info.num_subcores
    nw = nc * ns
    b_per_w = nrows // nw

    mesh = plsc.VectorSubcoreMesh(core_axis_name="c", subcore_axis_name="s")

    @functools.partial(
        pl.kernel,
        mesh=mesh,
        out_type=jax.ShapeDtypeStruct((nrows, PAD_DIM), jnp.float32),
        scratch_types=[
            pltpu.VMEM((b_per_w,), jnp.int32),
            pltpu.VMEM((2, CHUNK, PAD_DIM), jnp.float32),
            pltpu.VMEM_SHARED((VOCAB_PAD, PAD_DIM), jnp.float32),
            pltpu.SemaphoreType.DMA,
            pltpu.SemaphoreType.DMA,
            pltpu.SemaphoreType.DMA,
        ],
    )
    def gather_kernel(idx_hbm, table_hbm, out_hbm, idx_v, rows_v, table_sp,
                      gsem, wsem0, wsem1):
        sid = lax.axis_index("s")
        wid = sid * nc + lax.axis_index("c")
        base = wid * b_per_w
        n_chunks = b_per_w // CHUNK
        @pl.when(sid < 8)
        def _():
            pltpu.sync_copy(
                table_hbm.at[pl.ds(sid * (VOCAB_PAD // 8), VOCAB_PAD // 8)],
                table_sp.at[pl.ds(sid * (VOCAB_PAD // 8), VOCAB_PAD // 8)])
        pltpu.sync_copy(idx_hbm.at[pl.ds(offset + base, b_per_w)], idx_v)
        plsc.subcore_barrier()
        wsems = (wsem0, wsem1)
        writes = [None, None]
        for k in range(n_chunks):
            b = k % 2
            if writes[b] is not None:
                writes[b].wait()
            pltpu.async_copy(
                table_sp.at[idx_v.at[pl.ds(k * CHUNK, CHUNK)]],
                rows_v.at[b], gsem).wait()
            writes[b] = pltpu.async_copy(
                rows_v.at[b], out_hbm.at[pl.ds(base + k * CHUNK, CHUNK)],
                wsems[b])
        for w in writes:
            if w is not None:
                w.wait()

    return gather_kernel


def kernel(user_id, table):
    table_padded = jnp.pad(
        table, ((0, VOCAB_PAD - VOCAB), (0, PAD_DIM - EMBED_DIM)))
    half = BATCH // 2
    out_a = _build(0, half)(user_id, table_padded)
    out_b = _build(half, half)(user_id, table_padded)
    return jnp.concatenate(
        [out_a[:, :EMBED_DIM], out_b[:, :EMBED_DIM]], axis=0)


# speedup vs baseline: 1.3500x; 1.1613x over previous
import functools

import jax
import jax.numpy as jnp
from jax import lax
from jax.experimental import pallas as pl
from jax.experimental.pallas import tpu as pltpu
from jax.experimental.pallas import tpu_sc as plsc

VOCAB = 1000
VOCAB_PAD = 1024
EMBED_DIM = 32
BATCH = 16384
PAD_DIM = 128
CHUNK = 64
NBUF = 4


@functools.lru_cache(maxsize=None)
def _build():
    info = plsc.get_sparse_core_info()
    nc, ns = info.num_cores, info.num_subcores
    nw = nc * ns
    b_per_w = BATCH // nw
    rows_per_stager = VOCAB_PAD // ns

    mesh = plsc.VectorSubcoreMesh(core_axis_name="c", subcore_axis_name="s")

    @functools.partial(
        pl.kernel,
        mesh=mesh,
        out_type=jax.ShapeDtypeStruct((BATCH, PAD_DIM), jnp.float32),
        scratch_types=[
            pltpu.VMEM((b_per_w,), jnp.int32),
            pltpu.VMEM((NBUF, CHUNK, PAD_DIM), jnp.float32),
            pltpu.VMEM_SHARED((VOCAB_PAD, PAD_DIM), jnp.float32),
            pltpu.SemaphoreType.DMA,
            pltpu.SemaphoreType.DMA,
        ] + [pltpu.SemaphoreType.DMA] * NBUF,
    )
    def gather_kernel(idx_hbm, table_hbm, out_hbm, idx_v, rows_v, table_sp,
                      isem, gsem, *wsems):
        sid = lax.axis_index("s")
        wid = sid * nc + lax.axis_index("c")
        base = wid * b_per_w
        n_chunks = b_per_w // CHUNK
        idx_cp = pltpu.async_copy(
            idx_hbm.at[pl.ds(base, b_per_w)], idx_v, isem)
        pltpu.sync_copy(
            table_hbm.at[pl.ds(sid * rows_per_stager, rows_per_stager)],
            table_sp.at[pl.ds(sid * rows_per_stager, rows_per_stager)])
        idx_cp.wait()
        plsc.subcore_barrier()
        writes = [None] * NBUF
        for k in range(n_chunks):
            b = k % NBUF
            if writes[b] is not None:
                writes[b].wait()
            pltpu.async_copy(
                table_sp.at[idx_v.at[pl.ds(k * CHUNK, CHUNK)]],
                rows_v.at[b], gsem).wait()
            writes[b] = pltpu.async_copy(
                rows_v.at[b], out_hbm.at[pl.ds(base + k * CHUNK, CHUNK)],
                wsems[b])
        for w in writes:
            if w is not None:
                w.wait()

    return gather_kernel


def kernel(user_id, table):
    table_padded = jnp.pad(
        table, ((0, VOCAB_PAD - VOCAB), (0, PAD_DIM - EMBED_DIM)))
    out_padded = _build()(user_id, table_padded)
    return out_padded[:, :EMBED_DIM]


# speedup vs baseline: 1.3555x; 1.0041x over previous
import functools

import jax
import jax.numpy as jnp
from jax import lax
from jax.experimental import pallas as pl
from jax.experimental.pallas import tpu as pltpu
from jax.experimental.pallas import tpu_sc as plsc

VOCAB = 1000
VOCAB_PAD = 1024
EMBED_DIM = 32
BATCH = 16384
PAD_DIM = 128
CHUNK = 128
NBUF = 4


@functools.lru_cache(maxsize=None)
def _build():
    info = plsc.get_sparse_core_info()
    nc, ns = info.num_cores, info.num_subcores
    nw = nc * ns
    b_per_w = BATCH // nw
    rows_per_stager = VOCAB_PAD // ns

    mesh = plsc.VectorSubcoreMesh(core_axis_name="c", subcore_axis_name="s")

    @functools.partial(
        pl.kernel,
        mesh=mesh,
        out_type=jax.ShapeDtypeStruct((BATCH, PAD_DIM), jnp.float32),
        scratch_types=[
            pltpu.VMEM((b_per_w,), jnp.int32),
            pltpu.VMEM((NBUF, CHUNK, PAD_DIM), jnp.float32),
            pltpu.VMEM_SHARED((VOCAB_PAD, PAD_DIM), jnp.float32),
            pltpu.SemaphoreType.DMA,
            pltpu.SemaphoreType.DMA,
        ] + [pltpu.SemaphoreType.DMA] * NBUF,
    )
    def gather_kernel(idx_hbm, table_hbm, out_hbm, idx_v, rows_v, table_sp,
                      isem, gsem, *wsems):
        sid = lax.axis_index("s")
        wid = sid * nc + lax.axis_index("c")
        base = wid * b_per_w
        n_chunks = b_per_w // CHUNK
        idx_cp = pltpu.async_copy(
            idx_hbm.at[pl.ds(base, b_per_w)], idx_v, isem)
        pltpu.sync_copy(
            table_hbm.at[pl.ds(sid * rows_per_stager, rows_per_stager)],
            table_sp.at[pl.ds(sid * rows_per_stager, rows_per_stager)])
        idx_cp.wait()
        plsc.subcore_barrier()
        writes = [None] * NBUF
        for k in range(n_chunks):
            b = k % NBUF
            if writes[b] is not None:
                writes[b].wait()
            pltpu.async_copy(
                table_sp.at[idx_v.at[pl.ds(k * CHUNK, CHUNK)]],
                rows_v.at[b], gsem).wait()
            writes[b] = pltpu.async_copy(
                rows_v.at[b], out_hbm.at[pl.ds(base + k * CHUNK, CHUNK)],
                wsems[b])
        for w in writes:
            if w is not None:
                w.wait()

    return gather_kernel


def kernel(user_id, table):
    table_padded = jnp.pad(
        table, ((0, VOCAB_PAD - VOCAB), (0, PAD_DIM - EMBED_DIM)))
    out_padded = _build()(user_id, table_padded)
    return out_padded[:, :EMBED_DIM]
